# baseline (speedup 1.0000x reference)
import functools

import jax
import jax.numpy as jnp
from jax import lax
from jax.experimental import pallas as pl
from jax.experimental.pallas import tpu as pltpu
from jax.experimental.pallas import tpu_sc as plsc

N = 10000
E = 320000
D = 128
DH = D // 2

NC = 2
NS = 16
NW = NC * NS

CHUNK = 128
DCPT = 81
MCPT0 = 116
MCPT1 = 46
EPT = CHUNK * DCPT
E_PAD = EPT * NW
N_PAD = 10240
NPT = N_PAD // NS
PAD_DST = N_PAD - 1
NBUF = 1

_mesh = plsc.VectorSubcoreMesh(
    core_axis_name="c", subcore_axis_name="s", num_cores=NC, num_subcores=NS)


@functools.partial(
    pl.kernel,
    out_type=jax.ShapeDtypeStruct((NC, N_PAD), jnp.float32),
    mesh=_mesh,
    scratch_types=[
        pltpu.VMEM((DCPT, CHUNK), jnp.int32),
        pltpu.VMEM((DCPT, CHUNK), jnp.float32),
        pltpu.VMEM((NPT,), jnp.float32),
        pltpu.VMEM_SHARED((N_PAD,), jnp.float32),
    ],
)
def _deg_kernel(dst_hbm, ew_hbm, out_hbm, dst_v, ew_v, buf_v, deg_sh):
    c = lax.axis_index("c")
    s = lax.axis_index("s")
    wid = c * NS + s
    for i in range(NPT // 16):
        buf_v[pl.ds(i * 16, 16)] = jnp.zeros((16,), jnp.float32)
    pltpu.sync_copy(buf_v, deg_sh.at[pl.ds(s * NPT, NPT)])
    plsc.subcore_barrier()
    pltpu.sync_copy(dst_hbm.at[wid], dst_v)
    pltpu.sync_copy(ew_hbm.at[wid], ew_v)

    def body(j, carry):
        pltpu.sync_copy(ew_v.at[j], deg_sh.at[dst_v.at[j]], add=True)
        return carry

    lax.fori_loop(0, DCPT, body, None)
    plsc.subcore_barrier()
    pltpu.sync_copy(deg_sh.at[pl.ds(s * NPT, NPT)],
                    out_hbm.at[c, pl.ds(s * NPT, NPT)])


@functools.partial(
    pl.kernel,
    out_type=jax.ShapeDtypeStruct((NC, N_PAD, D), jnp.float32),
    mesh=_mesh,
    scratch_types=[
        pltpu.VMEM((MCPT0, CHUNK), jnp.int32),
        pltpu.VMEM((MCPT0, CHUNK), jnp.float32),
        pltpu.VMEM((CHUNK,), jnp.int32),
        pltpu.VMEM((CHUNK,), jnp.int32),
        pltpu.VMEM((CHUNK, D), jnp.float32),
        pltpu.SemaphoreType.DMA,
        pltpu.SemaphoreType.DMA,
        pltpu.VMEM_SHARED((N_PAD, D), jnp.float32),
    ],
    compiler_params=pltpu.CompilerParams(use_tc_tiling_on_sc=False),
)
def _msg_kernel(y_hbm, pk_hbm, ew_hbm, out_hbm,
                pk_v, ew_v, sidx, didx, rows, gsem, ssem, acc_sh):
    c = lax.axis_index("c")
    s = lax.axis_index("s")
    mcpt = jnp.where(c == 0, MCPT1, MCPT0)

    def zrow(r, carry):
        for c0 in range(0, D, 16):
            rows[r, pl.ds(c0, 16)] = jnp.zeros((16,), jnp.float32)
        return carry

    lax.fori_loop(0, CHUNK, zrow, None)
    for k in range(NPT // CHUNK):
        pltpu.sync_copy(rows, acc_sh.at[pl.ds(s * NPT + k * CHUNK, CHUNK)])
    plsc.subcore_barrier()

    wid = c * NS + s
    pltpu.sync_copy(pk_hbm.at[wid], pk_v)
    pltpu.sync_copy(ew_hbm.at[wid], ew_v)

    def start_gather(j):
        for c0 in range(0, CHUNK, 16):
            sidx[pl.ds(c0, 16)] = \
                (pk_v[j, pl.ds(c0, 16)] & 16383) + c * N
        pltpu.async_copy(y_hbm.at[sidx], rows, gsem)

    def wait_gather():
        pltpu.make_async_copy(y_hbm.at[pl.ds(0, CHUNK)], rows, gsem).wait()

    def start_scatter(j):
        for c0 in range(0, CHUNK, 16):
            didx[pl.ds(c0, 16)] = \
                lax.shift_right_logical(pk_v[j, pl.ds(c0, 16)], 14)
        pltpu.async_copy(rows, acc_sh.at[didx], ssem, add=True)

    def wait_scatter():
        pltpu.make_async_copy(rows, acc_sh.at[pl.ds(0, CHUNK)], ssem).wait()

    def scale_rows(j):
        def rbody(g, c2):
            r0 = g * 16
            wv = ew_v[j, pl.ds(r0, 16)]
            for l in range(16):
                w = wv[l]
                for c0 in range(0, D, 16):
                    rows[r0 + l, pl.ds(c0, 16)] = \
                        rows[r0 + l, pl.ds(c0, 16)] * w
            return c2

        lax.fori_loop(0, CHUNK // 16, rbody, None)

    start_gather(0)

    def body(j, carry):
        wait_gather()
        scale_rows(j)
        start_scatter(j)

        @pl.when(j + 1 < mcpt)
        def _refill():
            wait_scatter()
            start_gather(j + 1)

        return carry

    lax.fori_loop(0, mcpt, body, None)
    wait_scatter()
    plsc.subcore_barrier()
    pltpu.sync_copy(acc_sh.at[pl.ds(s * NPT, NPT)],
                    out_hbm.at[c, pl.ds(s * NPT, NPT)])


def _dinv_body(degp_ref, dinv_ref):
    deg = degp_ref[0, :] + degp_ref[1, :] + 1.0
    dinv_ref[...] = lax.rsqrt(deg).reshape(1, N_PAD)


def _tc_dinv(degp):
    return pl.pallas_call(
        _dinv_body,
        out_shape=jax.ShapeDtypeStruct((1, N_PAD), jnp.float32),
    )(degp)


def _mm_body(x_ref, w_ref, scale_ref, o_ref):
    o_ref[...] = jnp.dot(scale_ref[...] * x_ref[...], w_ref[...],
                         preferred_element_type=jnp.float32)


def _tc_scaled_mm(x, w, scale):
    return pl.pallas_call(
        _mm_body,
        out_shape=jax.ShapeDtypeStruct((x.shape[0], w.shape[1]), jnp.float32),
    )(x, w, scale)


def _bn(h, g, b):
    m = jnp.mean(h, axis=0, keepdims=True)
    v = jnp.mean((h - m) ** 2, axis=0, keepdims=True)
    return (h - m) * lax.rsqrt(v + 1e-5) * g + b


def _gcn_pre(sp_ref, y_ref, dinv_ref, b_ref):
    agg = sp_ref[0, :N, :] + sp_ref[1, :N, :]
    return dinv_ref[...] * (agg + y_ref[...]) + b_ref[...]


def _mid_body(sp_ref, y_ref, dinv_ref, b_ref, g_ref, be_ref, w_ref, o_ref):
    pre = _gcn_pre(sp_ref, y_ref, dinv_ref, b_ref)
    h = jax.nn.relu(_bn(pre, g_ref[...], be_ref[...]))
    o_ref[...] = jnp.dot(dinv_ref[...] * h, w_ref[...],
                         preferred_element_type=jnp.float32)


def _tc_mid(sp, y, dinv_col, b, g, be, w):
    return pl.pallas_call(
        _mid_body,
        out_shape=jax.ShapeDtypeStruct((N, D), jnp.float32),
    )(sp, y, dinv_col, b, g, be, w)


def _final_body(sp_ref, y_ref, dinv_ref, b2_ref, g2_ref, be2_ref,
                dist_ref, degf_ref, wd_ref, bd_ref, gd_ref, bed_ref,
                wg_ref, bg_ref, gg_ref, beg_ref, wm_ref, bm_ref, o_ref):
    pre = _gcn_pre(sp_ref, y_ref, dinv_ref, b2_ref)
    h = jax.nn.relu(_bn(pre, g2_ref[...], be2_ref[...]))
    d = jax.nn.relu(_bn(dist_ref[...] * wd_ref[...] + bd_ref[...],
                        gd_ref[...], bed_ref[...]))
    dg = jax.nn.relu(_bn(degf_ref[...] * wg_ref[...] + bg_ref[...],
                         gg_ref[...], beg_ref[...]))
    acc = jnp.dot(h, wm_ref[:D, :], preferred_element_type=jnp.float32)
    acc += jnp.dot(d, wm_ref[D:2 * D, :], preferred_element_type=jnp.float32)
    acc += jnp.dot(dg, wm_ref[2 * D:, :], preferred_element_type=jnp.float32)
    o_ref[...] = acc + bm_ref[...]


def _tc_final(sp, y, dinv_col, b2, g2, be2, dist, degf,
              wd, bd, gd, bed, wg, bg, gg, beg, wm, bm):
    return pl.pallas_call(
        _final_body,
        out_shape=jax.ShapeDtypeStruct((N, D), jnp.float32),
    )(sp, y, dinv_col, b2, g2, be2, dist, degf,
      wd, bd, gd, bed, wg, bg, gg, beg, wm, bm)


def kernel(x, edge_index, edge_weight, dist_feat, degree_feat,
           W1, b1, g1, be1, W2, b2, g2, be2, Wd, bd, gd, bed,
           Wg, bg, gg, beg, Wm, bm):
    src = edge_index[0].astype(jnp.int32)
    dst = edge_index[1].astype(jnp.int32)
    ew = edge_weight.astype(jnp.float32)

    pad = E_PAD - E
    srcp = jnp.concatenate([src, jnp.zeros((pad,), jnp.int32)])
    dstp = jnp.concatenate([dst, jnp.full((pad,), PAD_DST, jnp.int32)])
    ewp = jnp.concatenate([ew, jnp.zeros((pad,), jnp.float32)])
    dst32 = dstp.reshape(NW, DCPT, CHUNK)
    ew32 = ewp.reshape(NW, DCPT, CHUNK)

    pk = srcp | (dstp << 14)
    n0 = NS * MCPT0 * CHUNK
    pk_dead = PAD_DST << 14
    cpad = MCPT0 - MCPT1
    pk0 = pk[:n0].reshape(NS, MCPT0, CHUNK)
    pk1 = jnp.pad(pk[n0:].reshape(NS, MCPT1, CHUNK),
                  ((0, 0), (0, cpad), (0, 0)), constant_values=pk_dead)
    pk32 = jnp.concatenate([pk1, pk0])
    ew0 = ewp[:n0].reshape(NS, MCPT0, CHUNK)
    ew1 = jnp.pad(ewp[n0:].reshape(NS, MCPT1, CHUNK),
                  ((0, 0), (0, cpad), (0, 0)))
    ewm32 = jnp.concatenate([ew1, ew0])

    degp = _deg_kernel(dst32, ew32)
    dinv_row = _tc_dinv(degp)
    dinv_col = dinv_row.reshape(N_PAD, 1)[:N]

    y1 = _tc_scaled_mm(x, W1, dinv_col)
    y1_sc = jnp.concatenate([y1, y1])
    s1 = _msg_kernel(y1_sc, pk32, ewm32)
    y2 = _tc_mid(s1, y1, dinv_col, b1.reshape(1, D), g1.reshape(1, D),
                 be1.reshape(1, D), W2)
    y2_sc = jnp.concatenate([y2, y2])
    s2 = _msg_kernel(y2_sc, pk32, ewm32)
    out = _tc_final(s2, y2, dinv_col, b2.reshape(1, D), g2.reshape(1, D),
                    be2.reshape(1, D), dist_feat, degree_feat,
                    Wd, bd.reshape(1, D), gd.reshape(1, D), bed.reshape(1, D),
                    Wg, bg.reshape(1, D), gg.reshape(1, D), beg.reshape(1, D),
                    Wm, bm.reshape(1, D))
    return out


# speedup vs baseline: 1.4087x; 1.4087x over previous
import functools

import jax
import jax.numpy as jnp
from jax import lax
from jax.experimental import pallas as pl
from jax.experimental.pallas import tpu as pltpu
from jax.experimental.pallas import tpu_sc as plsc

N = 10000
E = 320000
D = 128
DH = D // 2

NC = 2
NS = 16
NW = NC * NS

CHUNK = 128
DCPT = 80
MCPT = 160
EPT = CHUNK * DCPT
E_PAD = EPT * NW
N_PAD = 10240
NPT = N_PAD // NS
PAD_DST = N_PAD - 1
NBUF = 2

_mesh = plsc.VectorSubcoreMesh(
    core_axis_name="c", subcore_axis_name="s", num_cores=NC, num_subcores=NS)


@functools.partial(
    pl.kernel,
    out_type=jax.ShapeDtypeStruct((NC, N_PAD), jnp.float32),
    mesh=_mesh,
    scratch_types=[
        pltpu.VMEM((DCPT, CHUNK), jnp.int32),
        pltpu.VMEM((DCPT, CHUNK), jnp.float32),
        pltpu.VMEM((NPT,), jnp.float32),
        pltpu.VMEM_SHARED((N_PAD,), jnp.float32),
    ],
)
def _deg_kernel(dst_hbm, ew_hbm, out_hbm, dst_v, ew_v, buf_v, deg_sh):
    c = lax.axis_index("c")
    s = lax.axis_index("s")
    wid = c * NS + s
    for i in range(NPT // 16):
        buf_v[pl.ds(i * 16, 16)] = jnp.zeros((16,), jnp.float32)
    pltpu.sync_copy(buf_v, deg_sh.at[pl.ds(s * NPT, NPT)])
    plsc.subcore_barrier()
    pltpu.sync_copy(dst_hbm.at[wid], dst_v)
    pltpu.sync_copy(ew_hbm.at[wid], ew_v)

    def body(j, carry):
        pltpu.sync_copy(ew_v.at[j], deg_sh.at[dst_v.at[j]], add=True)
        return carry

    lax.fori_loop(0, DCPT, body, None)
    plsc.subcore_barrier()
    pltpu.sync_copy(deg_sh.at[pl.ds(s * NPT, NPT)],
                    out_hbm.at[c, pl.ds(s * NPT, NPT)])


@functools.partial(
    pl.kernel,
    out_type=jax.ShapeDtypeStruct((NC, N_PAD, DH), jnp.float32),
    mesh=_mesh,
    scratch_types=[
        pltpu.VMEM((MCPT, CHUNK), jnp.int32),
        pltpu.VMEM((MCPT, CHUNK), jnp.int32),
        pltpu.VMEM((MCPT, CHUNK), jnp.float32),
        [pltpu.VMEM((CHUNK,), jnp.int32) for _ in range(NBUF)],
        [pltpu.VMEM((CHUNK, DH), jnp.float32) for _ in range(NBUF)],
        [pltpu.SemaphoreType.DMA for _ in range(NBUF)],
        [pltpu.SemaphoreType.DMA for _ in range(NBUF)],
        pltpu.VMEM_SHARED((N_PAD, DH), jnp.float32),
    ],
    compiler_params=pltpu.CompilerParams(use_tc_tiling_on_sc=False),
)
def _msg_kernel(y_hbm, src_hbm, dst_hbm, ew_hbm, out_hbm,
                src_v, dst_v, ew_v, sidx, rows, gsem, ssem, acc_sh):
    c = lax.axis_index("c")
    s = lax.axis_index("s")

    def zrow(r, carry):
        for c0 in range(0, DH, 16):
            rows[0][r, pl.ds(c0, 16)] = jnp.zeros((16,), jnp.float32)
        return carry

    lax.fori_loop(0, CHUNK, zrow, None)
    for k in range(NPT // CHUNK):
        pltpu.sync_copy(rows[0], acc_sh.at[pl.ds(s * NPT + k * CHUNK, CHUNK)])
    plsc.subcore_barrier()

    pltpu.sync_copy(src_hbm.at[s], src_v)
    pltpu.sync_copy(dst_hbm.at[s], dst_v)
    pltpu.sync_copy(ew_hbm.at[s], ew_v)

    def start_gather(j, b):
        for c0 in range(0, CHUNK, 16):
            sidx[b][pl.ds(c0, 16)] = src_v[j, pl.ds(c0, 16)] * 2 + c
        pltpu.async_copy(y_hbm.at[sidx[b]], rows[b], gsem[b])

    def wait_gather(j, b):
        pltpu.make_async_copy(y_hbm.at[pl.ds(0, CHUNK)], rows[b],
                              gsem[b]).wait()

    def start_scatter(j, b):
        pltpu.async_copy(rows[b], acc_sh.at[dst_v.at[j]], ssem[b], add=True)

    def wait_scatter(j, b):
        pltpu.make_async_copy(rows[b], acc_sh.at[pl.ds(0, CHUNK)],
                              ssem[b]).wait()

    def scale_rows(j, b):
        def rbody(g, c2):
            r0 = g * 16
            wv = ew_v[j, pl.ds(r0, 16)]
            for l in range(16):
                w = wv[l]
                for c0 in range(0, DH, 16):
                    rows[b][r0 + l, pl.ds(c0, 16)] = \
                        rows[b][r0 + l, pl.ds(c0, 16)] * w
            return c2

        lax.fori_loop(0, CHUNK // 16, rbody, None)

    for b in range(NBUF):
        start_gather(b, b)

    def body(m, carry):
        for bb in range(NBUF):
            j = m * NBUF + bb
            wait_gather(j, bb)
            scale_rows(j, bb)
            start_scatter(j, bb)
            jn = j + NBUF - 1
            bp = (bb - 1) % NBUF

            @pl.when(jnp.logical_and(j >= 1, jn < MCPT))
            def _refill():
                wait_scatter(j - 1, bp)
                start_gather(jn, bp)

        return carry

    lax.fori_loop(0, MCPT // NBUF, body, None)

    for b in range(NBUF):
        wait_scatter(MCPT - NBUF + b, b)
    plsc.subcore_barrier()
    pltpu.sync_copy(acc_sh.at[pl.ds(s * NPT, NPT)],
                    out_hbm.at[c, pl.ds(s * NPT, NPT)])


def _dinv_body(degp_ref, dinv_ref):
    deg = degp_ref[0, :] + degp_ref[1, :] + 1.0
    dinv_ref[...] = lax.rsqrt(deg).reshape(1, N_PAD)


def _tc_dinv(degp):
    return pl.pallas_call(
        _dinv_body,
        out_shape=jax.ShapeDtypeStruct((1, N_PAD), jnp.float32),
    )(degp)


def _mm_body(x_ref, w_ref, scale_ref, o_ref):
    o_ref[...] = jnp.dot(scale_ref[...] * x_ref[...], w_ref[...],
                         preferred_element_type=jnp.float32)


def _tc_scaled_mm(x, w, scale):
    return pl.pallas_call(
        _mm_body,
        out_shape=jax.ShapeDtypeStruct((x.shape[0], w.shape[1]), jnp.float32),
    )(x, w, scale)


def _bn(h, g, b):
    m = jnp.mean(h, axis=0, keepdims=True)
    v = jnp.mean((h - m) ** 2, axis=0, keepdims=True)
    return (h - m) * lax.rsqrt(v + 1e-5) * g + b


def _gcn_pre(sp_ref, y_ref, dinv_ref, b_ref):
    agg = jnp.concatenate([sp_ref[0, :N, :], sp_ref[1, :N, :]], axis=1)
    return dinv_ref[...] * (agg + y_ref[...]) + b_ref[...]


def _mid_body(sp_ref, y_ref, dinv_ref, b_ref, g_ref, be_ref, w_ref, o_ref):
    pre = _gcn_pre(sp_ref, y_ref, dinv_ref, b_ref)
    h = jax.nn.relu(_bn(pre, g_ref[...], be_ref[...]))
    o_ref[...] = jnp.dot(dinv_ref[...] * h, w_ref[...],
                         preferred_element_type=jnp.float32)


def _tc_mid(sp, y, dinv_col, b, g, be, w):
    return pl.pallas_call(
        _mid_body,
        out_shape=jax.ShapeDtypeStruct((N, D), jnp.float32),
    )(sp, y, dinv_col, b, g, be, w)


def _final_body(sp_ref, y_ref, dinv_ref, b2_ref, g2_ref, be2_ref,
                dist_ref, degf_ref, wd_ref, bd_ref, gd_ref, bed_ref,
                wg_ref, bg_ref, gg_ref, beg_ref, wm_ref, bm_ref, o_ref):
    pre = _gcn_pre(sp_ref, y_ref, dinv_ref, b2_ref)
    h = jax.nn.relu(_bn(pre, g2_ref[...], be2_ref[...]))
    d = jax.nn.relu(_bn(dist_ref[...] * wd_ref[...] + bd_ref[...],
                        gd_ref[...], bed_ref[...]))
    dg = jax.nn.relu(_bn(degf_ref[...] * wg_ref[...] + bg_ref[...],
                         gg_ref[...], beg_ref[...]))
    acc = jnp.dot(h, wm_ref[:D, :], preferred_element_type=jnp.float32)
    acc += jnp.dot(d, wm_ref[D:2 * D, :], preferred_element_type=jnp.float32)
    acc += jnp.dot(dg, wm_ref[2 * D:, :], preferred_element_type=jnp.float32)
    o_ref[...] = acc + bm_ref[...]


def _tc_final(sp, y, dinv_col, b2, g2, be2, dist, degf,
              wd, bd, gd, bed, wg, bg, gg, beg, wm, bm):
    return pl.pallas_call(
        _final_body,
        out_shape=jax.ShapeDtypeStruct((N, D), jnp.float32),
    )(sp, y, dinv_col, b2, g2, be2, dist, degf,
      wd, bd, gd, bed, wg, bg, gg, beg, wm, bm)


def kernel(x, edge_index, edge_weight, dist_feat, degree_feat,
           W1, b1, g1, be1, W2, b2, g2, be2, Wd, bd, gd, bed,
           Wg, bg, gg, beg, Wm, bm):
    src = edge_index[0].astype(jnp.int32)
    dst = edge_index[1].astype(jnp.int32)
    ew = edge_weight.astype(jnp.float32)

    pad = E_PAD - E
    srcp = jnp.concatenate([src, jnp.zeros((pad,), jnp.int32)])
    dstp = jnp.concatenate([dst, jnp.full((pad,), PAD_DST, jnp.int32)])
    ewp = jnp.concatenate([ew, jnp.zeros((pad,), jnp.float32)])
    src16 = srcp.reshape(NS, MCPT, CHUNK)
    dst16 = dstp.reshape(NS, MCPT, CHUNK)
    ew16 = ewp.reshape(NS, MCPT, CHUNK)
    dst32 = dstp.reshape(NW, DCPT, CHUNK)
    ew32 = ewp.reshape(NW, DCPT, CHUNK)

    degp = _deg_kernel(dst32, ew32)
    dinv_row = _tc_dinv(degp)
    dinv_col = dinv_row.reshape(N_PAD, 1)[:N]

    y1 = _tc_scaled_mm(x, W1, dinv_col)
    y1_sc = y1.reshape(N, 2, DH).reshape(2 * N, DH)
    s1 = _msg_kernel(y1_sc, src16, dst16, ew16)
    y2 = _tc_mid(s1, y1, dinv_col, b1.reshape(1, D), g1.reshape(1, D),
                 be1.reshape(1, D), W2)
    y2_sc = y2.reshape(N, 2, DH).reshape(2 * N, DH)
    s2 = _msg_kernel(y2_sc, src16, dst16, ew16)
    out = _tc_final(s2, y2, dinv_col, b2.reshape(1, D), g2.reshape(1, D),
                    be2.reshape(1, D), dist_feat, degree_feat,
                    Wd, bd.reshape(1, D), gd.reshape(1, D), bed.reshape(1, D),
                    Wg, bg.reshape(1, D), gg.reshape(1, D), beg.reshape(1, D),
                    Wm, bm.reshape(1, D))
    return out
